# baseline (speedup 1.0000x reference)
import jax
import jax.numpy as jnp
from jax import lax
from jax.experimental import pallas as pl
from jax.experimental.pallas import tpu as pltpu
from jax.experimental.pallas import tpu_sc as plsc

B = 16384
D = 64
NC, NS, L = 2, 16, 16
NW = NC * NS
C = B // NW
H = C // 2
NKH = H // L
NJ = D // L
EPS = 1e-5


def _sc_body(xt_hbm, c1_hbm, c4_hbm, c5_hbm, emb_hbm, wt_hbm, bm_hbm,
             lnw_hbm, lnb_hbm, out_hbm,
             x0_v, x1_v, c1_v, c4_v, c5_v, al_v, be_v, ga_v, dl_v,
             emb_v, wt_v, bm_v, lnw_v, lnb_v, wvec, wtab,
             sin0, sin1, sout0, sout1):
    wid = lax.axis_index("s") * NC + lax.axis_index("c")
    base = wid * C
    xh = (x0_v, x1_v)

    hin = [pltpu.async_copy(xt_hbm.at[:, pl.ds(base + h * H, H)], xh[h], s)
           for h, s in ((0, sin0), (1, sin1))]

    pltpu.sync_copy(c1_hbm.at[pl.ds(base, C)], c1_v)
    pltpu.sync_copy(c4_hbm.at[pl.ds(base, C)], c4_v)
    pltpu.sync_copy(c5_hbm.at[pl.ds(base, C)], c5_v)
    pltpu.sync_copy(emb_hbm, emb_v)
    pltpu.sync_copy(wt_hbm, wt_v)
    pltpu.sync_copy(bm_hbm, bm_v)
    pltpu.sync_copy(lnw_hbm, lnw_v)
    pltpu.sync_copy(lnb_hbm, lnb_v)

    w0 = [wt_v[0, pl.ds(j * L, L)] for j in range(NJ)]
    w1 = [wt_v[1, pl.ds(j * L, L)] for j in range(NJ)]
    bm = [bm_v[pl.ds(j * L, L)] for j in range(NJ)]
    lnw = [lnw_v[pl.ds(j * L, L)] for j in range(NJ)]
    lnb = [lnb_v[pl.ds(j * L, L)] for j in range(NJ)]
    e0 = [emb_v[0, pl.ds(j * L, L)] for j in range(NJ)]
    e1 = [emb_v[1, pl.ds(j * L, L)] for j in range(NJ)]

    def vsum(chunks):
        acc = chunks[0]
        for ch in chunks[1:]:
            acc = acc + ch
        return jnp.sum(acc)

    inv_n = 1.0 / D
    mw0 = vsum(w0) * inv_n
    mw1 = vsum(w1) * inv_n
    mb = vsum(bm) * inv_n
    A = vsum([w0[j] * w0[j] for j in range(NJ)]) * inv_n - mw0 * mw0
    Bq = vsum([w1[j] * w1[j] for j in range(NJ)]) * inv_n - mw1 * mw1
    C2 = 2.0 * (vsum([w0[j] * w1[j] for j in range(NJ)]) * inv_n - mw0 * mw1)
    D2 = 2.0 * (vsum([w0[j] * bm[j] for j in range(NJ)]) * inv_n - mw0 * mb)
    E2 = 2.0 * (vsum([w1[j] * bm[j] for j in range(NJ)]) * inv_n - mw1 * mb)
    F = vsum([bm[j] * bm[j] for j in range(NJ)]) * inv_n - mb * mb + EPS

    for j in range(NJ):
        sl = pl.ds(j * L, L)
        wvec[0, sl] = (w0[j] - mw0) * lnw[j]
        wvec[1, sl] = (w1[j] - mw1) * lnw[j]
        wvec[2, sl] = (bm[j] - mb) * lnw[j]
        wvec[3, sl] = e0[j] + lnb[j]
        wvec[4, sl] = e1[j] - e0[j]

    def mk_tab(d, _):
        dd = jnp.full((L,), d, jnp.int32)
        for i in range(5):
            ii = jnp.full((L,), i, jnp.int32)
            wtab[d, pl.ds(i * L, L)] = plsc.load_gather(wvec, [ii, dd])
        return 0

    lax.fori_loop(0, D, mk_tab, 0)

    def scal(k, _):
        sl = pl.ds(k * L, L)
        c4g = c4_v[sl]
        c5g = c5_v[sl]
        var = (A * c4g * c4g + Bq * c5g * c5g + C2 * c4g * c5g
               + D2 * c4g + E2 * c5g + F)
        i = lax.bitcast_convert_type(var, jnp.int32)
        i = 0x5F3759DF - lax.shift_right_arithmetic(i, 1)
        y = lax.bitcast_convert_type(i, jnp.float32)
        for _ in range(3):
            y = y * (1.5 - 0.5 * var * y * y)
        al_v[sl] = y * c4g
        be_v[sl] = y * c5g
        ga_v[sl] = y
        dl_v[sl] = c1_v[sl].astype(jnp.float32)
        return 0

    lax.fori_loop(0, C // L, scal, 0)

    hout = []
    for h, sin, sout in ((0, hin[0], sout0), (1, hin[1], sout1)):
        sin.wait()
        kbase = h * NKH
        xv = xh[h]

        def dloop(d2, _, kbase=kbase, xv=xv):
            dts = []
            for u in range(2):
                d = d2 * 2 + u
                dts.append([wtab[d, pl.ds(i * L, L)] for i in range(5)])

            def kloop(k, _, dts=dts, xv=xv):
                slg = pl.ds((kbase + k) * L, L)
                sl = pl.ds(k * L, L)
                al = al_v[slg]
                be = be_v[slg]
                ga = ga_v[slg]
                dl = dl_v[slg]
                for u in range(2):
                    d = d2 * 2 + u
                    uv, vv, twv, t2v, dlv = dts[u]
                    xv[d, sl] = (xv[d, sl] + t2v + dl * dlv
                                 + al * uv + be * vv + ga * twv)
                return 0

            lax.fori_loop(0, NKH, kloop, 0)
            return 0

        lax.fori_loop(0, D // 2, dloop, 0)
        hout.append(pltpu.async_copy(
            xv, out_hbm.at[:, pl.ds(base + h * H, H)], sout))
    for ho in hout:
        ho.wait()


def kernel(cond0, cond1, cond4, cond5, emb_table, W_meta, b_meta, ln_w, ln_b):
    mesh = plsc.VectorSubcoreMesh(core_axis_name="c", subcore_axis_name="s")
    f = pl.kernel(
        _sc_body,
        out_type=jax.ShapeDtypeStruct((D, B), jnp.float32),
        mesh=mesh,
        compiler_params=pltpu.CompilerParams(needs_layout_passes=False),
        scratch_types=[
            pltpu.VMEM((D, H), jnp.float32),
            pltpu.VMEM((D, H), jnp.float32),
            pltpu.VMEM((C,), jnp.int32),
            pltpu.VMEM((C,), jnp.float32),
            pltpu.VMEM((C,), jnp.float32),
            pltpu.VMEM((C,), jnp.float32),
            pltpu.VMEM((C,), jnp.float32),
            pltpu.VMEM((C,), jnp.float32),
            pltpu.VMEM((C,), jnp.float32),
            pltpu.VMEM((2, D), jnp.float32),
            pltpu.VMEM((2, D), jnp.float32),
            pltpu.VMEM((D,), jnp.float32),
            pltpu.VMEM((D,), jnp.float32),
            pltpu.VMEM((D,), jnp.float32),
            pltpu.VMEM((5, D), jnp.float32),
            pltpu.VMEM((D, 5 * L), jnp.float32),
            pltpu.SemaphoreType.DMA,
            pltpu.SemaphoreType.DMA,
            pltpu.SemaphoreType.DMA,
            pltpu.SemaphoreType.DMA,
        ],
    )
    out_t = f(cond0.T, cond1, cond4.reshape(B), cond5.reshape(B),
              emb_table, W_meta.T, b_meta, ln_w, ln_b)
    return out_t.T
